# baseline (speedup 1.0000x reference)
import jax
import jax.numpy as jnp
from jax.experimental import pallas as pl
from jax.experimental.pallas import tpu as pltpu

NUM_CLASSES = 1000
B = 64
K = 8
SUB = B // K
NSTEP = 4096 // B


def _zero3d(idx_ref, out_ref, scratch, sems):
    i = pl.program_id(0)
    slot = jax.lax.rem(i, 2)

    @pl.when(i >= 2)
    def _():
        for k in range(K):
            pltpu.make_async_copy(
                scratch.at[slot, pl.ds(k * SUB, SUB)],
                out_ref.at[pl.ds((i - 2) * B + k * SUB, SUB)],
                sems.at[slot, k],
            ).wait()

    scratch[slot] = jnp.zeros((B, 26, NUM_CLASSES), jnp.int32)
    for k in range(K):
        pltpu.make_async_copy(
            scratch.at[slot, pl.ds(k * SUB, SUB)],
            out_ref.at[pl.ds(i * B + k * SUB, SUB)],
            sems.at[slot, k],
        ).start()

    @pl.when(i == NSTEP - 1)
    def _():
        for k in range(K):
            pltpu.make_async_copy(
                scratch.at[1 - slot, pl.ds(k * SUB, SUB)],
                out_ref.at[pl.ds((i - 1) * B + k * SUB, SUB)],
                sems.at[1 - slot, k],
            ).wait()
        for k in range(K):
            pltpu.make_async_copy(
                scratch.at[slot, pl.ds(k * SUB, SUB)],
                out_ref.at[pl.ds(i * B + k * SUB, SUB)],
                sems.at[slot, k],
            ).wait()


def kernel(indices):
    rows, cols = indices.shape
    out = pl.pallas_call(
        _zero3d,
        grid=(NSTEP,),
        in_specs=[pl.BlockSpec((rows, cols), lambda i: (0, 0))],
        out_specs=pl.BlockSpec(memory_space=pl.ANY),
        out_shape=jax.ShapeDtypeStruct((rows, cols, NUM_CLASSES), jnp.int32),
        scratch_shapes=[
            pltpu.VMEM((2, B, 26, NUM_CLASSES), jnp.int32),
            pltpu.SemaphoreType.DMA((2, K)),
        ],
    )(indices)
    return out


# speedup vs baseline: 4.5790x; 4.5790x over previous
import jax
import jax.numpy as jnp
from jax.experimental import pallas as pl

NUM_CLASSES = 1000
KB = 200


def _onehot_block(idx_ref, out_ref):
    j = pl.program_id(1)
    k = jax.lax.broadcasted_iota(jnp.int32, (1, KB, 4096), 1) + j * KB
    out_ref[...] = (idx_ref[...] == k).astype(jnp.int32)


def kernel(indices):
    rows, cols = indices.shape
    idx_t = indices.T.reshape(cols, 1, rows)
    out = pl.pallas_call(
        _onehot_block,
        grid=(cols, NUM_CLASSES // KB),
        in_specs=[pl.BlockSpec((1, 1, rows), lambda c, j: (c, 0, 0))],
        out_specs=pl.BlockSpec((1, KB, rows), lambda c, j: (c, j, 0)),
        out_shape=jax.ShapeDtypeStruct((cols, NUM_CLASSES, rows), jnp.int32),
    )(idx_t)
    return out.transpose(2, 0, 1)


# speedup vs baseline: 4.6150x; 1.0079x over previous
import jax
import jax.numpy as jnp
from jax.experimental import pallas as pl

NUM_CLASSES = 1000
KB = 1000


def _onehot_block(idx_ref, out_ref):
    j = pl.program_id(1)
    k = jax.lax.broadcasted_iota(jnp.int32, (1, KB, 4096), 1) + j * KB
    out_ref[...] = (idx_ref[...] == k).astype(jnp.int32)


def kernel(indices):
    rows, cols = indices.shape
    idx_t = indices.T.reshape(cols, 1, rows)
    out = pl.pallas_call(
        _onehot_block,
        grid=(cols, NUM_CLASSES // KB),
        in_specs=[pl.BlockSpec((1, 1, rows), lambda c, j: (c, 0, 0))],
        out_specs=pl.BlockSpec((1, KB, rows), lambda c, j: (c, j, 0)),
        out_shape=jax.ShapeDtypeStruct((cols, NUM_CLASSES, rows), jnp.int32),
    )(idx_t)
    return out.transpose(2, 0, 1)
